# baseline (speedup 1.0000x reference)
import functools

import jax
import jax.numpy as jnp
from jax import lax
from jax.experimental import pallas as pl
from jax.experimental.pallas import tpu as pltpu
from jax.experimental.pallas import tpu_sc as plsc

_EMBED = 64
_MAXLEN = 200
_LABELS = 64
_BATCH = 4096

_NC, _NS = 2, 16
_NW = _NC * _NS
_BPW = _BATCH // _NW
_C0, _C1 = 104, 96


def _make_mean_kernel():
    mesh = plsc.VectorSubcoreMesh(core_axis_name="c", subcore_axis_name="s")

    @functools.partial(
        pl.kernel,
        out_type=jax.ShapeDtypeStruct((_BATCH, _EMBED), jnp.float32),
        mesh=mesh,
        compiler_params=pltpu.CompilerParams(use_tc_tiling_on_sc=False),
        scratch_types=[
            pltpu.VMEM((_BPW, _MAXLEN), jnp.int32),
            pltpu.VMEM((_MAXLEN, _EMBED), jnp.float32),
            pltpu.VMEM((_BPW, _EMBED), jnp.float32),
            pltpu.SemaphoreType.DMA,
        ],
    )
    def mean_kernel(x_hbm, table_hbm, out_hbm, idx_v, rows_v, out_v, sem):
        wid = lax.axis_index("s") * _NC + lax.axis_index("c")
        base = wid * _BPW
        pltpu.sync_copy(x_hbm.at[pl.ds(base, _BPW)], idx_v)

        inv_len = jnp.float32(1.0 / _MAXLEN)

        def batch_body(i, carry):
            cp0 = pltpu.async_copy(
                table_hbm.at[idx_v.at[i, pl.ds(0, _C0)]],
                rows_v.at[pl.ds(0, _C0)], sem)
            cp1 = pltpu.async_copy(
                table_hbm.at[idx_v.at[i, pl.ds(_C0, _C1)]],
                rows_v.at[pl.ds(_C0, _C1)], sem)
            cp0.wait()
            cp1.wait()

            def row_body(r, accs):
                return tuple(
                    accs[c] + rows_v[r, pl.ds(c * 16, 16)] for c in range(4)
                )

            accs = tuple(jnp.zeros((16,), jnp.float32) for _ in range(4))
            accs = lax.fori_loop(0, _MAXLEN, row_body, accs)
            for c in range(4):
                out_v[i, pl.ds(c * 16, 16)] = accs[c] * inv_len
            return carry

        lax.fori_loop(0, _BPW, batch_body, 0)
        pltpu.sync_copy(out_v, out_hbm.at[pl.ds(base, _BPW)])

    return mean_kernel


_MEAN_KERNEL = _make_mean_kernel()


def _fc_body(m_ref, w_ref, b_ref, o_ref):
    o_ref[...] = (
        jnp.dot(m_ref[...], w_ref[...], preferred_element_type=jnp.float32)
        + b_ref[...]
    )


def kernel(x, table, fc_w, fc_b):
    mean = _MEAN_KERNEL(x, table)
    out = pl.pallas_call(
        _fc_body,
        out_shape=jax.ShapeDtypeStruct((_BATCH, _LABELS), jnp.float32),
    )(mean, fc_w.T, fc_b.reshape(1, _LABELS))
    return out


# speedup vs baseline: 1.1712x; 1.1712x over previous
import functools

import jax
import jax.numpy as jnp
from jax import lax
from jax.experimental import pallas as pl
from jax.experimental.pallas import tpu as pltpu
from jax.experimental.pallas import tpu_sc as plsc

_EMBED = 64
_MAXLEN = 200
_LABELS = 64
_BATCH = 4096

_NC, _NS = 2, 16
_NW = _NC * _NS
_BPW = _BATCH // _NW
_C0, _C1 = 104, 96


def _make_mean_kernel():
    mesh = plsc.VectorSubcoreMesh(core_axis_name="c", subcore_axis_name="s")

    @functools.partial(
        pl.kernel,
        out_type=jax.ShapeDtypeStruct((_BATCH, _EMBED), jnp.float32),
        mesh=mesh,
        compiler_params=pltpu.CompilerParams(use_tc_tiling_on_sc=False),
        scratch_types=[
            pltpu.VMEM((_BPW, _MAXLEN), jnp.int32),
            pltpu.VMEM((2, _MAXLEN, _EMBED), jnp.float32),
            pltpu.VMEM((_BPW, _EMBED), jnp.float32),
            pltpu.SemaphoreType.DMA,
            pltpu.SemaphoreType.DMA,
        ],
    )
    def mean_kernel(x_hbm, table_hbm, out_hbm, idx_v, rows_v, out_v, sem0, sem1):
        wid = lax.axis_index("s") * _NC + lax.axis_index("c")
        base = wid * _BPW
        pltpu.sync_copy(x_hbm.at[pl.ds(base, _BPW)], idx_v)

        inv_len = jnp.float32(1.0 / _MAXLEN)

        def fire(row, buf, sem):
            pltpu.async_copy(
                table_hbm.at[idx_v.at[row, pl.ds(0, _C0)]],
                rows_v.at[buf, pl.ds(0, _C0)], sem)
            pltpu.async_copy(
                table_hbm.at[idx_v.at[row, pl.ds(_C0, _C1)]],
                rows_v.at[buf, pl.ds(_C0, _C1)], sem)

        def wait(buf, sem):
            pltpu.make_async_copy(
                table_hbm.at[idx_v.at[0, pl.ds(0, _C0)]],
                rows_v.at[buf, pl.ds(0, _C0)], sem).wait()
            pltpu.make_async_copy(
                table_hbm.at[idx_v.at[0, pl.ds(_C0, _C1)]],
                rows_v.at[buf, pl.ds(_C0, _C1)], sem).wait()

        def accum_store(i, buf):
            def row_body(rr, accs):
                r = rr * 8
                for u in range(8):
                    accs = tuple(
                        accs[c] + rows_v[buf, r + u, pl.ds(c * 16, 16)]
                        for c in range(4)
                    )
                return accs

            accs = tuple(jnp.zeros((16,), jnp.float32) for _ in range(4))
            accs = lax.fori_loop(0, _MAXLEN // 8, row_body, accs)
            for c in range(4):
                out_v[i, pl.ds(c * 16, 16)] = accs[c] * inv_len

        fire(0, 0, sem0)

        def pair_body(ii, carry):
            i0 = ii * 2
            fire(i0 + 1, 1, sem1)
            wait(0, sem0)
            accum_store(i0, 0)
            nxt = jnp.where(i0 + 2 < _BPW, i0 + 2, 0)
            fire(nxt, 0, sem0)
            wait(1, sem1)
            accum_store(i0 + 1, 1)
            return carry

        lax.fori_loop(0, _BPW // 2, pair_body, 0)
        wait(0, sem0)
        pltpu.sync_copy(out_v, out_hbm.at[pl.ds(base, _BPW)])

    return mean_kernel


_MEAN_KERNEL = _make_mean_kernel()


def _fc_body(m_ref, w_ref, b_ref, o_ref):
    o_ref[...] = (
        jnp.dot(m_ref[...], w_ref[...], preferred_element_type=jnp.float32)
        + b_ref[...]
    )


def kernel(x, table, fc_w, fc_b):
    mean = _MEAN_KERNEL(x, table)
    out = pl.pallas_call(
        _fc_body,
        out_shape=jax.ShapeDtypeStruct((_BATCH, _LABELS), jnp.float32),
    )(mean, fc_w.T, fc_b.reshape(1, _LABELS))
    return out
